# Initial kernel scaffold; baseline (speedup 1.0000x reference)
import functools

import jax
import jax.numpy as jnp
from jax import lax
from jax.experimental import pallas as pl
from jax.experimental.pallas import tpu as pltpu
from jax.experimental.pallas import tpu_sc as plsc

B = 4096
L = 200
DIM = 64
N = B * L
NW = 32
ROWS_W = N // NW
CHUNK = 512
NCHUNK = ROWS_W // CHUNK
IPG = 128
GPC = CHUNK // IPG
LANES = 16

_mesh = plsc.VectorSubcoreMesh(core_axis_name="c", subcore_axis_name="s")


@functools.partial(
    pl.kernel,
    mesh=_mesh,
    out_type=jax.ShapeDtypeStruct((NW, NCHUNK, CHUNK, DIM), jnp.float32),
    scratch_types=[
        pltpu.VMEM((GPC, IPG), jnp.int32),
        pltpu.VMEM((CHUNK, DIM), jnp.float32),
        pltpu.VMEM((CHUNK, DIM), jnp.float32),
        pltpu.SemaphoreType.DMA,
        pltpu.SemaphoreType.DMA,
    ],
)
def _pe_kernel(x_hbm, idx_hbm, tab_hbm, out_hbm, idx_v, rows_v, xb_v, gsem, xsem):
    wid = lax.axis_index("s") * 2 + lax.axis_index("c")

    def chunk_body(c, carry):
        pltpu.sync_copy(idx_hbm.at[wid, c], idx_v)
        for j in range(GPC):
            pltpu.async_copy(
                tab_hbm.at[idx_v.at[j]],
                rows_v.at[pl.ds(j * IPG, IPG)],
                gsem,
            )
        xcp = pltpu.async_copy(x_hbm.at[wid, c], xb_v, xsem)
        for j in range(GPC):
            pltpu.make_async_copy(
                tab_hbm.at[idx_v.at[j]],
                rows_v.at[pl.ds(j * IPG, IPG)],
                gsem,
            ).wait()
        xcp.wait()

        def add_body(r, carry2):
            for k in range(DIM // LANES):
                sl = pl.ds(k * LANES, LANES)
                plsc.addupdate(xb_v.at[r, sl], rows_v[r, sl])
            return carry2

        lax.fori_loop(0, CHUNK, add_body, 0, unroll=2)
        pltpu.sync_copy(xb_v, out_hbm.at[wid, c])
        return carry

    lax.fori_loop(0, NCHUNK, chunk_body, 0)


def kernel(x, order, pos_enc):
    x_r = x.reshape(NW, NCHUNK, CHUNK, DIM)
    idx_r = order.astype(jnp.int32).reshape(NW, NCHUNK, GPC, IPG)
    out = _pe_kernel(x_r, idx_r, pos_enc)
    return out.reshape(B, L, DIM)


# speedup vs baseline: 2.2276x; 2.2276x over previous
import functools

import jax
import jax.numpy as jnp
from jax import lax
from jax.experimental import pallas as pl
from jax.experimental.pallas import tpu as pltpu
from jax.experimental.pallas import tpu_sc as plsc

B = 4096
L = 200
DIM = 64
N = B * L
NW = 32
ROWS_W = N // NW
CHUNK = 512
NCHUNK = ROWS_W // CHUNK
IPG = 128
GPC = CHUNK // IPG
LANES = 16

_mesh = plsc.VectorSubcoreMesh(core_axis_name="c", subcore_axis_name="s")


@functools.partial(
    pl.kernel,
    mesh=_mesh,
    compiler_params=pltpu.CompilerParams(use_tc_tiling_on_sc=False),
    out_type=jax.ShapeDtypeStruct((NW, NCHUNK, CHUNK, DIM), jnp.float32),
    scratch_types=[
        pltpu.VMEM((GPC, IPG), jnp.int32),
        pltpu.VMEM((CHUNK, DIM), jnp.float32),
        pltpu.VMEM((CHUNK, DIM), jnp.float32),
        pltpu.SemaphoreType.DMA,
        pltpu.SemaphoreType.DMA,
    ],
)
def _pe_kernel(x_hbm, idx_hbm, tab_hbm, out_hbm, idx_v, rows_v, xb_v, gsem, xsem):
    wid = lax.axis_index("s") * 2 + lax.axis_index("c")

    def chunk_body(c, carry):
        pltpu.sync_copy(idx_hbm.at[wid, c], idx_v)
        for j in range(GPC):
            pltpu.async_copy(
                tab_hbm.at[idx_v.at[j]],
                rows_v.at[pl.ds(j * IPG, IPG)],
                gsem,
            )
        xcp = pltpu.async_copy(x_hbm.at[wid, c], xb_v, xsem)
        for j in range(GPC):
            pltpu.make_async_copy(
                tab_hbm.at[idx_v.at[j]],
                rows_v.at[pl.ds(j * IPG, IPG)],
                gsem,
            ).wait()
        xcp.wait()

        def add_body(r, carry2):
            for k in range(DIM // LANES):
                sl = pl.ds(k * LANES, LANES)
                plsc.addupdate(xb_v.at[r, sl], rows_v[r, sl])
            return carry2

        lax.fori_loop(0, CHUNK, add_body, 0, unroll=2)
        pltpu.sync_copy(xb_v, out_hbm.at[wid, c])
        return carry

    lax.fori_loop(0, NCHUNK, chunk_body, 0)


def kernel(x, order, pos_enc):
    x_r = x.reshape(NW, NCHUNK, CHUNK, DIM)
    idx_r = order.astype(jnp.int32).reshape(NW, NCHUNK, GPC, IPG)
    out = _pe_kernel(x_r, idx_r, pos_enc)
    return out.reshape(B, L, DIM)


# speedup vs baseline: 2.5310x; 1.1362x over previous
import functools

import jax
import jax.numpy as jnp
from jax import lax
from jax.experimental import pallas as pl
from jax.experimental.pallas import tpu as pltpu
from jax.experimental.pallas import tpu_sc as plsc

B = 4096
L = 200
DIM = 64
N = B * L
NW = 32
ROWS_W = N // NW
CHUNK = 128
NCHUNK = ROWS_W // CHUNK
NSLOT = 4
LANES = 16

_mesh = plsc.VectorSubcoreMesh(core_axis_name="c", subcore_axis_name="s")


@functools.partial(
    pl.kernel,
    mesh=_mesh,
    compiler_params=pltpu.CompilerParams(use_tc_tiling_on_sc=False),
    out_type=jax.ShapeDtypeStruct((NW, NCHUNK, CHUNK, DIM), jnp.float32),
    scratch_types=[
        pltpu.VMEM((NCHUNK, CHUNK), jnp.int32),
        pltpu.VMEM((NSLOT, CHUNK, DIM), jnp.float32),
        pltpu.VMEM((NSLOT, CHUNK, DIM), jnp.float32),
        pltpu.SemaphoreType.DMA,
        pltpu.SemaphoreType.DMA,
        pltpu.SemaphoreType.DMA,
        pltpu.SemaphoreType.DMA,
        pltpu.SemaphoreType.DMA,
        pltpu.SemaphoreType.DMA,
        pltpu.SemaphoreType.DMA,
        pltpu.SemaphoreType.DMA,
    ],
)
def _pe_kernel(x_hbm, idx_hbm, tab_hbm, out_hbm, idx_all, rows_v, xb_v,
               l0, l1, l2, l3, o0, o1, o2, o3):
    lsem = (l0, l1, l2, l3)
    osem = (o0, o1, o2, o3)
    wid = lax.axis_index("s") * 2 + lax.axis_index("c")

    pltpu.sync_copy(idx_hbm.at[wid], idx_all)

    def load(c, s):
        pltpu.async_copy(tab_hbm.at[idx_all.at[c]], rows_v.at[s], lsem[s])
        pltpu.async_copy(x_hbm.at[wid, c], xb_v.at[s], lsem[s])

    def wait_loads(s):
        pltpu.make_async_copy(tab_hbm.at[idx_all.at[0]], rows_v.at[s], lsem[s]).wait()
        pltpu.make_async_copy(x_hbm.at[wid, 0], xb_v.at[s], lsem[s]).wait()

    def wait_out(s):
        pltpu.make_async_copy(xb_v.at[s], out_hbm.at[wid, 0], osem[s]).wait()

    for s in range(NSLOT - 1):
        load(s, s)

    def chunk_group(p, carry):
        c0 = p * NSLOT
        for s in range(NSLOT):
            c = c0 + s
            wait_loads(s)

            def add_body(r, carry2):
                for k in range(DIM // LANES):
                    sl = pl.ds(k * LANES, LANES)
                    plsc.addupdate(xb_v.at[s, r, sl], rows_v[s, r, sl])
                return carry2

            lax.fori_loop(0, CHUNK, add_body, 0, unroll=4)
            pltpu.async_copy(xb_v.at[s], out_hbm.at[wid, c], osem[s])

            cn = c + NSLOT - 1
            sn = (s + NSLOT - 1) % NSLOT

            @pl.when(cn < NCHUNK)
            def _():
                @pl.when(cn >= NSLOT)
                def _():
                    wait_out(sn)

                load(cn, sn)

        return carry

    lax.fori_loop(0, NCHUNK // NSLOT, chunk_group, 0)

    for s in range(NSLOT):
        wait_out(s)


def kernel(x, order, pos_enc):
    x_r = x.reshape(NW, NCHUNK, CHUNK, DIM)
    idx_r = order.astype(jnp.int32).reshape(NW, NCHUNK, CHUNK)
    out = _pe_kernel(x_r, idx_r, pos_enc)
    return out.reshape(B, L, DIM)
